# baseline (speedup 1.0000x reference)
import functools

import jax
import jax.numpy as jnp
from jax import lax
from jax.experimental import pallas as pl
from jax.experimental.pallas import tpu as pltpu
from jax.experimental.pallas import tpu_sc as plsc

N = 10000
E = 320000
D = 128
L = 3

NP = 10240
NTILES = 16
NCORES = 2
NW = NTILES * NCORES
EC = 64
NCH = 160
EPT = NCH * EC
EPAD = NW * EPT
RPT = NP // NTILES
NB = 2
NSLOT = 2 * NB
SLAB = 16
NSLAB = NCH // SLAB
BR = 1280


def _sc_body(m_hbm, src_hbm, dst_hbm, out_hbm, src_sl, dst_sl, buf_v, acc_sh,
             gsem, ssem, isem):
    cid = lax.axis_index("c")
    sid = lax.axis_index("s")

    zeros16 = jnp.zeros((16,), jnp.float32)

    def zrow(i, carry):
        r = i // (D // 16)
        c0 = (i % (D // 16)) * 16
        buf_v[0, r, pl.ds(c0, 16)] = zeros16
        return carry

    lax.fori_loop(0, EC * (D // 16), zrow, 0)

    row0 = sid * RPT

    def zacc(k, carry):
        pltpu.sync_copy(buf_v.at[0], acc_sh.at[pl.ds(row0 + k * EC, EC)])
        return carry

    lax.fori_loop(0, RPT // EC, zacc, 0)

    pltpu.sync_copy(src_hbm.at[cid, sid, 0], src_sl.at[0])
    pltpu.sync_copy(dst_hbm.at[cid, sid, 0], dst_sl.at[0])
    pltpu.async_copy(src_hbm.at[cid, sid, 1], src_sl.at[1], isem)
    pltpu.async_copy(dst_hbm.at[cid, sid, 1], dst_sl.at[1], isem)
    for j in range(NB):
        pltpu.async_copy(m_hbm.at[src_sl.at[0, j]], buf_v.at[j], gsem.at[j])

    plsc.subcore_barrier()

    def wait_islab(s):
        hn = (s + 1) % 3
        pltpu.make_async_copy(src_hbm.at[cid, sid, s + 1], src_sl.at[hn],
                              isem).wait()
        pltpu.make_async_copy(dst_hbm.at[cid, sid, s + 1], dst_sl.at[hn],
                              isem).wait()

    def visit(s, j, first):
        hc = s % 3
        b = j % NSLOT
        bg = (j + NB) % NSLOT
        pltpu.make_async_copy(m_hbm.at[src_sl.at[hc, j]], buf_v.at[b],
                              gsem.at[b]).wait()
        pltpu.async_copy(buf_v.at[b], acc_sh.at[dst_sl.at[hc, j]],
                         ssem.at[b], add=True)
        if not (first and j < NB):
            pltpu.make_async_copy(buf_v.at[bg], acc_sh.at[dst_sl.at[hc, j]],
                                  ssem.at[bg]).wait()
        if j < SLAB - NB:
            pltpu.async_copy(m_hbm.at[src_sl.at[hc, j + NB]], buf_v.at[bg],
                             gsem.at[bg])
        else:
            hn = (s + 1) % 3

            @pl.when(s < NSLAB - 1)
            def _():
                pltpu.async_copy(m_hbm.at[src_sl.at[hn, j + NB - SLAB]],
                                 buf_v.at[bg], gsem.at[bg])

    for j in range(SLAB):
        if j == SLAB - NB:
            wait_islab(0)
        visit(0, j, first=True)

    def slab(s, carry):
        hn = (s + 1) % 3

        @pl.when(s < NSLAB - 1)
        def _():
            pltpu.async_copy(src_hbm.at[cid, sid, s + 1], src_sl.at[hn], isem)
            pltpu.async_copy(dst_hbm.at[cid, sid, s + 1], dst_sl.at[hn], isem)
        for j in range(SLAB):
            if j == SLAB - NB:
                @pl.when(s < NSLAB - 1)
                def _():
                    wait_islab(s)
            visit(s, j, first=False)
        return carry

    lax.fori_loop(1, NSLAB, slab, 0)

    for k in range(NB):
        b = (NCH - NB + k) % NSLOT
        pltpu.make_async_copy(buf_v.at[b],
                              acc_sh.at[dst_sl.at[(NSLAB - 1) % 3,
                                                  SLAB - NB + k]],
                              ssem.at[b]).wait()

    plsc.subcore_barrier()
    pltpu.sync_copy(acc_sh.at[pl.ds(row0, RPT)],
                    out_hbm.at[cid, pl.ds(row0, RPT)])


_sc_aggregate = functools.partial(
    pl.kernel,
    mesh=plsc.VectorSubcoreMesh(core_axis_name="c", subcore_axis_name="s"),
    compiler_params=pltpu.CompilerParams(use_tc_tiling_on_sc=False),
    out_type=jax.ShapeDtypeStruct((NCORES, NP, D), jnp.float32),
    scratch_types=[
        pltpu.VMEM((3, SLAB, EC), jnp.int32),
        pltpu.VMEM((3, SLAB, EC), jnp.int32),
        pltpu.VMEM((NSLOT, EC, D), jnp.float32),
        pltpu.VMEM_SHARED((NP, D), jnp.float32),
        pltpu.SemaphoreType.DMA((NSLOT,)),
        pltpu.SemaphoreType.DMA((NSLOT,)),
        pltpu.SemaphoreType.DMA,
    ],
)(_sc_body)


def _tc_init_body(x_ref, w0t_ref, g0_ref, x1_ref, m_ref):
    x1 = jax.nn.sigmoid(
        jnp.dot(x_ref[...], w0t_ref[...], preferred_element_type=jnp.float32))
    x1_ref[...] = x1
    m_ref[...] = jnp.dot(x1, g0_ref[...], preferred_element_type=jnp.float32)


def _gru(p_ref, h_ref, wih_ref, whh_ref, bih_ref, bhh_ref):
    agg = p_ref[0] + p_ref[1]
    h = h_ref[...]
    gi = jnp.dot(agg, wih_ref[...], preferred_element_type=jnp.float32) + bih_ref[...]
    gh = jnp.dot(h, whh_ref[...], preferred_element_type=jnp.float32) + bhh_ref[...]
    r = jax.nn.sigmoid(gi[:, :D] + gh[:, :D])
    z = jax.nn.sigmoid(gi[:, D:2 * D] + gh[:, D:2 * D])
    n = jnp.tanh(gi[:, 2 * D:] + r * gh[:, 2 * D:])
    return (1.0 - z) * n + z * h


def _tc_gru_body(p_ref, h_ref, wih_ref, whh_ref, bih_ref, bhh_ref, wg_ref,
                 hn_ref, m_ref):
    hn = _gru(p_ref, h_ref, wih_ref, whh_ref, bih_ref, bhh_ref)
    hn_ref[...] = hn
    m_ref[...] = jnp.dot(hn, wg_ref[...], preferred_element_type=jnp.float32)


def _tc_final_body(p_ref, h_ref, wih_ref, whh_ref, bih_ref, bhh_ref, w1_ref,
                   b1_ref, out_ref):
    hn = _gru(p_ref, h_ref, wih_ref, whh_ref, bih_ref, bhh_ref)
    y = jnp.dot(jax.nn.relu(hn), w1_ref[...], preferred_element_type=jnp.float32)
    out_ref[...] = y + b1_ref[...]


_row_spec = pl.BlockSpec((BR, D), lambda i: (i, 0))
_p_spec = pl.BlockSpec((NCORES, BR, D), lambda i: (0, i, 0))


def _full(shape):
    return pl.BlockSpec(shape, lambda i: tuple(0 for _ in shape))


_tc_init = pl.pallas_call(
    _tc_init_body,
    grid=(NP // BR,),
    in_specs=[_row_spec, _full((D, D)), _full((D, D))],
    out_specs=[_row_spec, _row_spec],
    out_shape=[jax.ShapeDtypeStruct((NP, D), jnp.float32)] * 2,
)

_gru_specs = [_p_spec, _row_spec, _full((D, 3 * D)), _full((D, 3 * D)),
              _full((1, 3 * D)), _full((1, 3 * D))]

_tc_gru = pl.pallas_call(
    _tc_gru_body,
    grid=(NP // BR,),
    in_specs=_gru_specs + [_full((D, D))],
    out_specs=[_row_spec, _row_spec],
    out_shape=[jax.ShapeDtypeStruct((NP, D), jnp.float32)] * 2,
)

_tc_final = pl.pallas_call(
    _tc_final_body,
    grid=(NP // BR,),
    in_specs=_gru_specs + [_full((D, D)), _full((1, D))],
    out_specs=_row_spec,
    out_shape=jax.ShapeDtypeStruct((NP, D), jnp.float32),
)


def kernel(x, edge_index, lin0_w, ggc_w, w_ih, w_hh, b_ih, b_hh, lin1_w, lin1_b):
    xp = jnp.zeros((NP, D), jnp.float32).at[:N].set(x)
    src = edge_index[0]
    dst = edge_index[1]
    src_b = jnp.concatenate(
        [src, jnp.zeros((EPAD - E,), jnp.int32)]).reshape(
            NCORES, NTILES, NSLAB, SLAB, EC)
    dst_b = jnp.concatenate(
        [dst, jnp.full((EPAD - E,), NP - 1, jnp.int32)]).reshape(
            NCORES, NTILES, NSLAB, SLAB, EC)

    w_ihT = w_ih.T
    w_hhT = w_hh.T
    bih2 = b_ih.reshape(1, 3 * D)
    bhh2 = b_hh.reshape(1, 3 * D)
    w1p = jnp.zeros((D, D), jnp.float32).at[:, 0].set(lin1_w[0])
    b1p = jnp.full((1, D), lin1_b[0], jnp.float32)

    x1p, m = _tc_init(xp, lin0_w.T, ggc_w[0])
    h = x1p
    for i in range(L):
        partials = _sc_aggregate(m, src_b, dst_b)
        if i < L - 1:
            h, m = _tc_gru(partials, h, w_ihT, w_hhT, bih2, bhh2, ggc_w[i + 1])
        else:
            outf = _tc_final(partials, h, w_ihT, w_hhT, bih2, bhh2, w1p, b1p)
    return (outf[:N, 0], x1p[:N])


# speedup vs baseline: 1.3146x; 1.3146x over previous
import functools

import jax
import jax.numpy as jnp
from jax import lax
from jax.experimental import pallas as pl
from jax.experimental.pallas import tpu as pltpu
from jax.experimental.pallas import tpu_sc as plsc

N = 10000
E = 320000
D = 128
L = 3

DH = D // 2
NP = 10240
NTILES = 16
NCORES = 2
EC = 128
NCH = 160
EPT = NCH * EC
EPAD = NTILES * EPT
RPT = NP // NTILES
NBUF = 5
BR = 1280


def _sc_body(m_hbm, src_hbm, dst_hbm, out_hbm, src_v, dst_v, buf_v, acc_sh,
             g0, g1, g2, g3, g4):
    gsems = (g0, g1, g2, g3, g4)
    cid = lax.axis_index("c")
    sid = lax.axis_index("s")

    zeros16 = jnp.zeros((16,), jnp.float32)

    def zrow(i, carry):
        r = i // (DH // 16)
        c0 = (i % (DH // 16)) * 16
        buf_v[0, r, pl.ds(c0, 16)] = zeros16
        return carry

    lax.fori_loop(0, EC * (DH // 16), zrow, 0)

    row0 = sid * RPT

    def zacc(k, carry):
        pltpu.sync_copy(buf_v.at[0], acc_sh.at[pl.ds(row0 + k * EC, EC)])
        return carry

    lax.fori_loop(0, RPT // EC, zacc, 0)

    pltpu.sync_copy(src_hbm.at[cid, sid], src_v)
    pltpu.sync_copy(dst_hbm.at[sid], dst_v)

    plsc.subcore_barrier()

    for b in range(NBUF):
        pltpu.async_copy(m_hbm.at[src_v.at[b]], buf_v.at[b], gsems[b])

    def group(g, carry):
        for b in range(NBUF):
            c = g * NBUF + b
            pltpu.make_async_copy(m_hbm.at[src_v.at[c]], buf_v.at[b],
                                  gsems[b]).wait()
            pltpu.sync_copy(buf_v.at[b], acc_sh.at[dst_v.at[c]], add=True)
            cn = c + NBUF

            @pl.when(cn < NCH)
            def _():
                pltpu.async_copy(m_hbm.at[src_v.at[cn]], buf_v.at[b], gsems[b])
        return carry

    lax.fori_loop(0, NCH // NBUF, group, 0)

    plsc.subcore_barrier()
    pltpu.sync_copy(acc_sh.at[pl.ds(row0, RPT)],
                    out_hbm.at[cid, pl.ds(row0, RPT)])


_sc_aggregate = functools.partial(
    pl.kernel,
    mesh=plsc.VectorSubcoreMesh(core_axis_name="c", subcore_axis_name="s"),
    compiler_params=pltpu.CompilerParams(use_tc_tiling_on_sc=False),
    out_type=jax.ShapeDtypeStruct((NCORES, NP, DH), jnp.float32),
    scratch_types=[
        pltpu.VMEM((NCH, EC), jnp.int32),
        pltpu.VMEM((NCH, EC), jnp.int32),
        pltpu.VMEM((NBUF, EC, DH), jnp.float32),
        pltpu.VMEM_SHARED((NP, DH), jnp.float32),
        pltpu.SemaphoreType.DMA,
        pltpu.SemaphoreType.DMA,
        pltpu.SemaphoreType.DMA,
        pltpu.SemaphoreType.DMA,
        pltpu.SemaphoreType.DMA,
    ],
)(_sc_body)


def _split_cols(m2_ref, m):
    m2_ref[0] = m[:, :DH]
    m2_ref[1] = m[:, DH:]


def _tc_init_body(x_ref, w0t_ref, g0_ref, x1_ref, m2_ref):
    x1 = jax.nn.sigmoid(
        jnp.dot(x_ref[...], w0t_ref[...], preferred_element_type=jnp.float32))
    x1_ref[...] = x1
    _split_cols(m2_ref, jnp.dot(x1, g0_ref[...],
                                preferred_element_type=jnp.float32))


def _gru(p_ref, h_ref, wih_ref, whh_ref, bih_ref, bhh_ref):
    agg = jnp.concatenate([p_ref[0], p_ref[1]], axis=1)
    h = h_ref[...]
    gi = jnp.dot(agg, wih_ref[...], preferred_element_type=jnp.float32) + bih_ref[...]
    gh = jnp.dot(h, whh_ref[...], preferred_element_type=jnp.float32) + bhh_ref[...]
    r = jax.nn.sigmoid(gi[:, :D] + gh[:, :D])
    z = jax.nn.sigmoid(gi[:, D:2 * D] + gh[:, D:2 * D])
    n = jnp.tanh(gi[:, 2 * D:] + r * gh[:, 2 * D:])
    return (1.0 - z) * n + z * h


def _tc_gru_body(p_ref, h_ref, wih_ref, whh_ref, bih_ref, bhh_ref, wg_ref,
                 hn_ref, m2_ref):
    hn = _gru(p_ref, h_ref, wih_ref, whh_ref, bih_ref, bhh_ref)
    hn_ref[...] = hn
    _split_cols(m2_ref, jnp.dot(hn, wg_ref[...],
                                preferred_element_type=jnp.float32))


def _tc_final_body(p_ref, h_ref, wih_ref, whh_ref, bih_ref, bhh_ref, w1_ref,
                   b1_ref, out_ref):
    hn = _gru(p_ref, h_ref, wih_ref, whh_ref, bih_ref, bhh_ref)
    y = jnp.dot(jax.nn.relu(hn), w1_ref[...], preferred_element_type=jnp.float32)
    out_ref[...] = y + b1_ref[...]


_row_spec = pl.BlockSpec((BR, D), lambda i: (i, 0))
_p_spec = pl.BlockSpec((NCORES, BR, DH), lambda i: (0, i, 0))
_m2_spec = pl.BlockSpec((NCORES, BR, DH), lambda i: (0, i, 0))
_m2_shape = jax.ShapeDtypeStruct((NCORES, NP, DH), jnp.float32)


def _full(shape):
    return pl.BlockSpec(shape, lambda i: tuple(0 for _ in shape))


_tc_init = pl.pallas_call(
    _tc_init_body,
    grid=(NP // BR,),
    in_specs=[_row_spec, _full((D, D)), _full((D, D))],
    out_specs=[_row_spec, _m2_spec],
    out_shape=[jax.ShapeDtypeStruct((NP, D), jnp.float32), _m2_shape],
)

_gru_specs = [_p_spec, _row_spec, _full((D, 3 * D)), _full((D, 3 * D)),
              _full((1, 3 * D)), _full((1, 3 * D))]

_tc_gru = pl.pallas_call(
    _tc_gru_body,
    grid=(NP // BR,),
    in_specs=_gru_specs + [_full((D, D))],
    out_specs=[_row_spec, _m2_spec],
    out_shape=[jax.ShapeDtypeStruct((NP, D), jnp.float32), _m2_shape],
)

_tc_final = pl.pallas_call(
    _tc_final_body,
    grid=(NP // BR,),
    in_specs=_gru_specs + [_full((D, D)), _full((1, D))],
    out_specs=_row_spec,
    out_shape=jax.ShapeDtypeStruct((NP, D), jnp.float32),
)


def kernel(x, edge_index, lin0_w, ggc_w, w_ih, w_hh, b_ih, b_hh, lin1_w, lin1_b):
    xp = jnp.zeros((NP, D), jnp.float32).at[:N].set(x)
    src = edge_index[0]
    dst = edge_index[1]
    src_t = jnp.concatenate(
        [src, jnp.zeros((EPAD - E,), jnp.int32)]).reshape(NTILES, NCH, EC)
    src_b = jnp.stack([src_t, src_t + NP])
    dst_b = jnp.concatenate(
        [dst, jnp.full((EPAD - E,), NP - 1, jnp.int32)]).reshape(
            NTILES, NCH, EC)

    w_ihT = w_ih.T
    w_hhT = w_hh.T
    bih2 = b_ih.reshape(1, 3 * D)
    bhh2 = b_hh.reshape(1, 3 * D)
    w1p = jnp.zeros((D, D), jnp.float32).at[:, 0].set(lin1_w[0])
    b1p = jnp.full((1, D), lin1_b[0], jnp.float32)

    x1p, m2 = _tc_init(xp, lin0_w.T, ggc_w[0])
    h = x1p
    for i in range(L):
        partials = _sc_aggregate(m2.reshape(NCORES * NP, DH), src_b, dst_b)
        if i < L - 1:
            h, m2 = _tc_gru(partials, h, w_ihT, w_hhT, bih2, bhh2, ggc_w[i + 1])
        else:
            outf = _tc_final(partials, h, w_ihT, w_hhT, bih2, bhh2, w1p, b1p)
    return (outf[:N, 0], x1p[:N])
